# baseline (speedup 1.0000x reference)
import functools
import jax
import jax.numpy as jnp
from jax import lax
from jax.experimental import pallas as pl
from jax.experimental.pallas import tpu as pltpu
from jax.experimental.pallas import tpu_sc as plsc

_N = 10000
_E = 320000
_NC = 2
_NS = 16
_NW = _NC * _NS
_CW = 250
_ERows = _E // _CW
_EPT = _E // _NW
_EPS = _E // _NS

_f32 = jnp.float32
_i32 = jnp.int32

_SC_PARAMS = pltpu.CompilerParams(
    needs_layout_passes=False, use_tc_tiling_on_sc=False)


def _mesh():
    return plsc.VectorSubcoreMesh(
        core_axis_name="c", subcore_axis_name="s",
        num_cores=_NC, num_subcores=_NS)


def _sc_scores(ab, src_flat, dst_flat):
    @functools.partial(
        pl.kernel,
        out_type=[jax.ShapeDtypeStruct((_E,), _f32),
                  jax.ShapeDtypeStruct((_NW * _N,), _f32)],
        mesh=_mesh(),
        compiler_params=_SC_PARAMS,
        scratch_types=[
            pltpu.VMEM((2 * _N,), _f32),
            pltpu.VMEM((_EPT,), _i32),
            pltpu.VMEM((_EPT,), _i32),
            pltpu.VMEM((_EPT,), _f32),
            pltpu.VMEM((_N,), _f32),
            pltpu.SemaphoreType.DMA,
        ],
    )
    def k(ab_hbm, src_hbm, dst_hbm, ex_hbm, den_hbm,
          ab_v, src_v, dst_v, ex_v, den_v, sem):
        cid = lax.axis_index("c")
        sid = lax.axis_index("s")
        wid = sid * _NC + cid
        ebase = wid * _EPT
        pltpu.sync_copy(ab_hbm, ab_v)
        pltpu.sync_copy(src_hbm.at[pl.ds(ebase, _EPT)], src_v)
        pltpu.sync_copy(dst_hbm.at[pl.ds(ebase, _EPT)], dst_v)

        zf = jnp.zeros((16,), _f32)
        def zbody(i, _):
            den_v[pl.ds(i * 16, 16)] = zf
            return ()
        lax.fori_loop(0, _N // 16, zbody, ())

        def ebody(g, _):
            s16 = src_v[pl.ds(g * 16, 16)]
            d16 = dst_v[pl.ds(g * 16, 16)]
            a = plsc.load_gather(ab_v, [s16 * 2])
            b = plsc.load_gather(ab_v, [d16 * 2 + 1])
            s = a + b
            s = jnp.where(s >= 0.0, s, s * 0.2)
            e16 = jnp.exp(s)
            ex_v[pl.ds(g * 16, 16)] = e16
            plsc.addupdate_scatter(den_v, [d16], e16)
            return ()
        lax.fori_loop(0, _EPT // 16, ebody, ())

        pltpu.sync_copy(ex_v, ex_hbm.at[pl.ds(ebase, _EPT)])
        pltpu.sync_copy(den_v, den_hbm.at[pl.ds(wid * _N, _N)])

    ex, den = k(ab.reshape(2 * _N), src_flat, dst_flat)
    return ex, den.reshape(_NW, _N).T


def _sc_msg3(tabs, src2d, dst2d, ex_flat):
    D = 80
    CWM = 125
    nchunk = _EPS // CWM
    nvr = D // 16

    @functools.partial(
        pl.kernel,
        out_type=jax.ShapeDtypeStruct((3 * _NC * _N, D), _f32),
        mesh=_mesh(),
        compiler_params=_SC_PARAMS,
        scratch_types=[
            pltpu.VMEM((CWM, D), _f32),
            pltpu.VMEM((CWM, D), _f32),
            pltpu.VMEM((nchunk, CWM), _i32),
            pltpu.VMEM((nchunk, CWM), _i32),
            pltpu.VMEM((_EPS,), _f32),
            pltpu.VMEM_SHARED((_N, D), _f32),
            pltpu.SemaphoreType.DMA,
            pltpu.SemaphoreType.DMA,
        ],
    )
    def k(fa, fb, q0, q1, q2, q3, src_hbm, dst_hbm, ex_hbm, out_hbm,
          rows_v, rows_w, src_v, dst_v, ex_v, acc, semA, semB):
        cid = lax.axis_index("c")
        sid = lax.axis_index("s")
        rbase = sid * nchunk
        pltpu.sync_copy(src_hbm.at[pl.ds(rbase, nchunk)], src_v)
        pltpu.sync_copy(dst_hbm.at[pl.ds(rbase, nchunk)], dst_v)
        pltpu.sync_copy(ex_hbm.at[pl.ds(sid * _EPS, _EPS)], ex_v)

        zf = jnp.zeros((16,), _f32)
        def zrow(r, _):
            for kk in range(nvr):
                rows_v[r, pl.ds(kk * 16, 16)] = zf
            return ()

        def sweep(tab_hbm):
            def gather(c, buf, sem):
                pltpu.async_copy(tab_hbm.at[src_v.at[c]], buf, sem)

            def process(c, buf, sem):
                pltpu.make_async_copy(
                    tab_hbm.at[src_v.at[c]], buf, sem).wait()
                def scale(r, _):
                    ev = plsc.load_gather(
                        ex_v, [jnp.full((16,), c * CWM + r, _i32)])
                    for kk in range(nvr):
                        sl = pl.ds(kk * 16, 16)
                        buf[r, sl] = buf[r, sl] * ev
                    return ()
                lax.fori_loop(0, CWM, scale, ())
                pltpu.sync_copy(buf, acc.at[dst_v.at[c]], add=True)

            gather(0, rows_v, semA)
            def pair(i, _):
                c0 = 2 * i
                gather(c0 + 1, rows_w, semB)
                process(c0, rows_v, semA)
                @pl.when(c0 + 2 < nchunk)
                def _():
                    gather(c0 + 2, rows_v, semA)
                process(c0 + 1, rows_w, semB)
                return ()
            lax.fori_loop(0, nchunk // 2, pair, ())

        for p, (ta, tb) in enumerate(((fa, fb), (q0, q1), (q2, q3))):
            lax.fori_loop(0, CWM, zrow, ())
            base = sid * 625
            for kk in range(5):
                pltpu.sync_copy(rows_v, acc.at[pl.ds(base + kk * CWM, CWM)])
            plsc.subcore_barrier()
            @pl.when(cid == 0)
            def _(ta=ta):
                sweep(ta)
            @pl.when(cid == 1)
            def _(tb=tb):
                sweep(tb)
            plsc.subcore_barrier()
            obase = p * 2 * _N + cid * _N + sid * 625
            pltpu.sync_copy(acc.at[pl.ds(sid * 625, 625)],
                            out_hbm.at[pl.ds(obase, 625)])

    out = k(*tabs, src2d.reshape(_EPS * _NS // CWM, CWM),
            dst2d.reshape(_EPS * _NS // CWM, CWM), ex_flat)
    return out.reshape(3, _NC, _N, 80)


def _sc_pq_gather(p_tab, q_tab, src_flat, dst_flat):
    cwc = 40
    nchunk = _EPT // cwc
    bf16 = jnp.bfloat16

    @functools.partial(
        pl.kernel,
        out_type=jax.ShapeDtypeStruct((_E, 512), bf16),
        mesh=_mesh(),
        compiler_params=_SC_PARAMS,
        scratch_types=[
            pltpu.VMEM((cwc, 512), bf16),
            pltpu.VMEM((cwc, 512), bf16),
            pltpu.VMEM((cwc, 512), bf16),
            pltpu.VMEM((cwc, 512), bf16),
            pltpu.VMEM((_EPT,), _i32),
            pltpu.VMEM((_EPT,), _i32),
            pltpu.SemaphoreType.DMA,
            pltpu.SemaphoreType.DMA,
        ],
    )
    def k(p_hbm, q_hbm, src_hbm, dst_hbm, g_hbm,
          pb0, qb0, pb1, qb1, src_v, dst_v, sem0, sem1):
        cid = lax.axis_index("c")
        sid = lax.axis_index("s")
        wid = sid * _NC + cid
        ebase = wid * _EPT
        pltpu.sync_copy(src_hbm.at[pl.ds(ebase, _EPT)], src_v)
        pltpu.sync_copy(dst_hbm.at[pl.ds(ebase, _EPT)], dst_v)

        def gather(c, pbuf, qbuf, sem):
            d1 = pltpu.async_copy(
                p_hbm.at[dst_v.at[pl.ds(c * cwc, cwc)]], pbuf, sem)
            d2 = pltpu.async_copy(
                q_hbm.at[src_v.at[pl.ds(c * cwc, cwc)]], qbuf, sem)
            return d1, d2

        def process(c, pbuf, qbuf, sem):
            pltpu.make_async_copy(
                p_hbm.at[dst_v.at[pl.ds(c * cwc, cwc)]], pbuf, sem).wait()
            pltpu.make_async_copy(
                q_hbm.at[src_v.at[pl.ds(c * cwc, cwc)]], qbuf, sem).wait()
            def addrow(r, _):
                for kk in range(16):
                    sl = pl.ds(kk * 32, 32)
                    pbuf[r, sl] = pbuf[r, sl] + qbuf[r, sl]
                return ()
            lax.fori_loop(0, cwc, addrow, ())
            pltpu.sync_copy(pbuf, g_hbm.at[pl.ds(ebase + c * cwc, cwc)])

        gather(0, pb0, qb0, sem0)
        def pair(i, _):
            c0 = 2 * i
            gather(c0 + 1, pb1, qb1, sem1)
            process(c0, pb0, qb0, sem0)
            @pl.when(c0 + 2 < nchunk)
            def _():
                gather(c0 + 2, pb0, qb0, sem0)
            process(c0 + 1, pb1, qb1, sem1)
            return ()
        lax.fori_loop(0, nchunk // 2, pair, ())

    return k(p_tab, q_tab, src_flat, dst_flat)


def _tc_ab(feat, wac):
    def body(x_ref, w_ref, o_ref):
        o_ref[...] = jnp.dot(x_ref[...], w_ref[...],
                             preferred_element_type=_f32)
    return pl.pallas_call(
        body,
        grid=(_N // 1000,),
        in_specs=[pl.BlockSpec((1000, 128), lambda i: (i, 0)),
                  pl.BlockSpec((128, 2), lambda i: (0, 0))],
        out_specs=pl.BlockSpec((1000, 2), lambda i: (i, 0)),
        out_shape=jax.ShapeDtypeStruct((_N, 2), _f32),
    )(feat, wac)


def _tc_node_update(nf_lo, nf_hi, nlq, den, mf, ml01, ml23,
                    fcWa, fcWb, fcb, fclWx, fclWm, fclb, proj_ws, final):
    NB = 400

    def body(nflo_ref, nfhi_ref, q0_ref, q1_ref, q2_ref, q3_ref, den_ref,
             mf_ref, ml01_ref, ml23_ref,
             fcWa_ref, fcWb_ref, fcb_ref, fclWx_ref, fclWm_ref, fclb_ref,
             *rest):
        if final:
            wdf_ref, wdl_ref, wsl_ref, wsf_ref, cb1_ref, p_ref, q_ref = rest
        else:
            (wa_ref, nflo_o, nfhi_o, q0_o, q1_o, q2_o, q3_o, ab_o) = rest
        den = jnp.sum(den_ref[...], axis=1) + 1e-9
        inv = (1.0 / den)[:, None]
        mfn = jnp.concatenate([mf_ref[0, :, :64], mf_ref[1, :, :64]],
                              axis=1) * inv
        mln = jnp.concatenate([ml01_ref[0], ml01_ref[1],
                               ml23_ref[0], ml23_ref[1]], axis=1) * inv
        nf_in = jnp.concatenate([nflo_ref[:, :64], nfhi_ref[:, :64]], axis=1)
        nl_in = jnp.concatenate([q0_ref[...], q1_ref[...],
                                 q2_ref[...], q3_ref[...]], axis=1)
        nf2 = jnp.dot(nf_in, fcWa_ref[...], preferred_element_type=_f32)
        nf2 = nf2 + jnp.dot(mfn, fcWb_ref[...], preferred_element_type=_f32)
        nf2 = jnp.maximum(nf2 + fcb_ref[...], 0.0)
        nl2 = jnp.dot(nl_in, fclWx_ref[...], preferred_element_type=_f32)
        nl2 = nl2 + jnp.dot(mln, fclWm_ref[...], preferred_element_type=_f32)
        nl2 = jnp.maximum(nl2 + fclb_ref[...], 0.0)
        if final:
            p = jnp.dot(nf2, wdf_ref[...], preferred_element_type=_f32)
            p = p + jnp.dot(nl2, wdl_ref[...], preferred_element_type=_f32)
            p_ref[...] = (p + cb1_ref[...]).astype(jnp.bfloat16)
            q = jnp.dot(nl2, wsl_ref[...], preferred_element_type=_f32)
            q = q + jnp.dot(nf2, wsf_ref[...], preferred_element_type=_f32)
            q_ref[...] = q.astype(jnp.bfloat16)
        else:
            zpad = jnp.zeros((nf2.shape[0], 16), _f32)
            nflo_o[...] = jnp.concatenate([nf2[:, :64], zpad], axis=1)
            nfhi_o[...] = jnp.concatenate([nf2[:, 64:], zpad], axis=1)
            q0_o[...] = nl2[:, 0:80]
            q1_o[...] = nl2[:, 80:160]
            q2_o[...] = nl2[:, 160:240]
            q3_o[...] = nl2[:, 240:320]
            ab_o[...] = jnp.dot(nf2, wa_ref[...], preferred_element_type=_f32)

    full = lambda shp: pl.BlockSpec(shp, lambda i: tuple(0 for _ in shp))
    in_specs = [
        pl.BlockSpec((NB, 80), lambda i: (i, 0)),
        pl.BlockSpec((NB, 80), lambda i: (i, 0)),
        pl.BlockSpec((NB, 80), lambda i: (i, 0)),
        pl.BlockSpec((NB, 80), lambda i: (i, 0)),
        pl.BlockSpec((NB, 80), lambda i: (i, 0)),
        pl.BlockSpec((NB, 80), lambda i: (i, 0)),
        pl.BlockSpec((NB, _NW), lambda i: (i, 0)),
        pl.BlockSpec((2, NB, 80), lambda i: (0, i, 0)),
        pl.BlockSpec((2, NB, 80), lambda i: (0, i, 0)),
        pl.BlockSpec((2, NB, 80), lambda i: (0, i, 0)),
        full((128, 128)), full((128, 128)), full((1, 128)),
        full((320, 320)), full((320, 320)), full((1, 320)),
    ]
    args = [nf_lo, nf_hi, *nlq, den, mf, ml01, ml23,
            fcWa, fcWb, fcb, fclWx, fclWm, fclb]
    if final:
        wdf, wdl, wsl, wsf, cb1 = proj_ws
        in_specs += [full((128, 512)), full((320, 512)), full((320, 512)),
                     full((128, 512)), full((1, 512))]
        args += [wdf, wdl, wsl, wsf, cb1]
        out_specs = [pl.BlockSpec((NB, 512), lambda i: (i, 0)),
                     pl.BlockSpec((NB, 512), lambda i: (i, 0))]
        out_shape = [jax.ShapeDtypeStruct((_N, 512), jnp.bfloat16),
                     jax.ShapeDtypeStruct((_N, 512), jnp.bfloat16)]
    else:
        wac = proj_ws
        in_specs += [full((128, 2))]
        args += [wac]
        out_specs = [pl.BlockSpec((NB, 80), lambda i: (i, 0)),
                     pl.BlockSpec((NB, 80), lambda i: (i, 0)),
                     pl.BlockSpec((NB, 80), lambda i: (i, 0)),
                     pl.BlockSpec((NB, 80), lambda i: (i, 0)),
                     pl.BlockSpec((NB, 80), lambda i: (i, 0)),
                     pl.BlockSpec((NB, 80), lambda i: (i, 0)),
                     pl.BlockSpec((NB, 2), lambda i: (i, 0))]
        out_shape = [jax.ShapeDtypeStruct((_N, 80), _f32),
                     jax.ShapeDtypeStruct((_N, 80), _f32),
                     jax.ShapeDtypeStruct((_N, 80), _f32),
                     jax.ShapeDtypeStruct((_N, 80), _f32),
                     jax.ShapeDtypeStruct((_N, 80), _f32),
                     jax.ShapeDtypeStruct((_N, 80), _f32),
                     jax.ShapeDtypeStruct((_N, 2), _f32)]

    return pl.pallas_call(
        body,
        grid=(_N // NB,),
        in_specs=in_specs,
        out_specs=out_specs,
        out_shape=out_shape,
    )(*args)


def _tc_readout(g, spatial, wsp, w2, cb2):
    EB = 2000

    def body(g_ref, sp_ref, wsp_ref, w2_ref, cb2_ref, o_ref):
        h = g_ref[...].astype(_f32) + jnp.dot(sp_ref[...], wsp_ref[...],
                                              preferred_element_type=_f32)
        h = jnp.maximum(h, 0.0)
        o_ref[...] = jnp.dot(h, w2_ref[...],
                             preferred_element_type=_f32) + cb2_ref[...]

    return pl.pallas_call(
        body,
        grid=(_E // EB,),
        in_specs=[pl.BlockSpec((EB, 512), lambda i: (i, 0)),
                  pl.BlockSpec((EB, 16), lambda i: (i, 0)),
                  pl.BlockSpec((16, 512), lambda i: (0, 0)),
                  pl.BlockSpec((512, 117), lambda i: (0, 0)),
                  pl.BlockSpec((1, 117), lambda i: (0, 0))],
        out_specs=pl.BlockSpec((EB, 117), lambda i: (i, 0)),
        out_shape=jax.ShapeDtypeStruct((_E, 117), _f32),
    )(g, spatial, wsp, w2, cb2)


def kernel(feat, word2vec, spatial_feat, edge_index, Wa1, Wa2, fc_W, fc_b,
           fcl_W, fcl_b, cls_W1, cls_b1, cls_W2, cls_b2):
    src = edge_index[0]
    dst = edge_index[1]
    src2d = src.reshape(_ERows, _CW)
    dst2d = dst.reshape(_ERows, _CW)

    wa1c = jnp.concatenate([Wa1[:128], Wa1[128:]], axis=1)
    wa2c = jnp.concatenate([Wa2[:128], Wa2[128:]], axis=1)
    fcWa, fcWb = fc_W[:128], fc_W[128:]
    fcb = fc_b.reshape(1, 128)
    fclWx = jnp.pad(fcl_W[:300], ((0, 20), (0, 20)))
    fclWm = jnp.pad(fcl_W[300:], ((0, 20), (0, 20)))
    fclb = jnp.pad(fcl_b, (0, 20)).reshape(1, 320)
    w2v_p = jnp.pad(word2vec, ((0, 0), (0, 20)))
    w2vq = tuple(w2v_p[:, 80 * i:80 * (i + 1)] for i in range(4))
    feat_lo = jnp.pad(feat[:, :64], ((0, 0), (0, 16)))
    feat_hi = jnp.pad(feat[:, 64:], ((0, 0), (0, 16)))
    wdf = cls_W1[0:128]
    wdl = jnp.pad(cls_W1[128:428], ((0, 20), (0, 0)))
    wsp = cls_W1[428:444]
    wsl = jnp.pad(cls_W1[444:744], ((0, 20), (0, 0)))
    wsf = cls_W1[744:872]
    cb1 = cls_b1.reshape(1, 512)
    cb2 = cls_b2.reshape(1, 117)

    ab1 = _tc_ab(feat, wa1c)
    ex1, den1 = _sc_scores(ab1, src, dst)
    msg1 = _sc_msg3((feat_lo, feat_hi, *w2vq), src2d, dst2d, ex1)
    nf_lo, nf_hi, q0, q1, q2, q3, ab2 = _tc_node_update(
        feat_lo, feat_hi, w2vq, den1, msg1[0], msg1[1], msg1[2],
        fcWa, fcWb, fcb, fclWx, fclWm, fclb, wa2c, final=False)

    ex2, den2 = _sc_scores(ab2, src, dst)
    msg2 = _sc_msg3((nf_lo, nf_hi, q0, q1, q2, q3), src2d, dst2d, ex2)
    p_tab, q_tab = _tc_node_update(
        nf_lo, nf_hi, (q0, q1, q2, q3), den2, msg2[0], msg2[1], msg2[2],
        fcWa, fcWb, fcb, fclWx, fclWm, fclb,
        (wdf, wdl, wsl, wsf, cb1), final=True)

    g = _sc_pq_gather(p_tab, q_tab, src, dst)
    return _tc_readout(g, spatial_feat, wsp, cls_W2, cb2)


# speedup vs baseline: 1.0233x; 1.0233x over previous
import functools
import jax
import jax.numpy as jnp
from jax import lax
from jax.experimental import pallas as pl
from jax.experimental.pallas import tpu as pltpu
from jax.experimental.pallas import tpu_sc as plsc

_N = 10000
_E = 320000
_NC = 2
_NS = 16
_NW = _NC * _NS
_CW = 250
_ERows = _E // _CW
_EPT = _E // _NW
_EPS = _E // _NS

_f32 = jnp.float32
_i32 = jnp.int32

_SC_PARAMS = pltpu.CompilerParams(
    needs_layout_passes=False, use_tc_tiling_on_sc=False)


def _mesh():
    return plsc.VectorSubcoreMesh(
        core_axis_name="c", subcore_axis_name="s",
        num_cores=_NC, num_subcores=_NS)


def _sc_scores(ab, src_flat, dst_flat):
    @functools.partial(
        pl.kernel,
        out_type=[jax.ShapeDtypeStruct((_E,), _f32),
                  jax.ShapeDtypeStruct((_NW * _N,), _f32)],
        mesh=_mesh(),
        compiler_params=_SC_PARAMS,
        scratch_types=[
            pltpu.VMEM((2 * _N,), _f32),
            pltpu.VMEM((_EPT,), _i32),
            pltpu.VMEM((_EPT,), _i32),
            pltpu.VMEM((_EPT,), _f32),
            pltpu.VMEM((_N,), _f32),
            pltpu.SemaphoreType.DMA,
        ],
    )
    def k(ab_hbm, src_hbm, dst_hbm, ex_hbm, den_hbm,
          ab_v, src_v, dst_v, ex_v, den_v, sem):
        cid = lax.axis_index("c")
        sid = lax.axis_index("s")
        wid = sid * _NC + cid
        ebase = wid * _EPT
        pltpu.sync_copy(ab_hbm, ab_v)
        pltpu.sync_copy(src_hbm.at[pl.ds(ebase, _EPT)], src_v)
        pltpu.sync_copy(dst_hbm.at[pl.ds(ebase, _EPT)], dst_v)

        zf = jnp.zeros((16,), _f32)
        def zbody(i, _):
            den_v[pl.ds(i * 16, 16)] = zf
            return ()
        lax.fori_loop(0, _N // 16, zbody, ())

        def ebody(g, _):
            s16 = src_v[pl.ds(g * 16, 16)]
            d16 = dst_v[pl.ds(g * 16, 16)]
            a = plsc.load_gather(ab_v, [s16 * 2])
            b = plsc.load_gather(ab_v, [d16 * 2 + 1])
            s = a + b
            s = jnp.where(s >= 0.0, s, s * 0.2)
            e16 = jnp.exp(s)
            ex_v[pl.ds(g * 16, 16)] = e16
            plsc.addupdate_scatter(den_v, [d16], e16)
            return ()
        lax.fori_loop(0, _EPT // 16, ebody, ())

        pltpu.sync_copy(ex_v, ex_hbm.at[pl.ds(ebase, _EPT)])
        pltpu.sync_copy(den_v, den_hbm.at[pl.ds(wid * _N, _N)])

    ex, den = k(ab.reshape(2 * _N), src_flat, dst_flat)
    return ex, den.reshape(_NW, 25, 400)


def _sc_msg(tab_a, tab_b, src2d, dst2d, ex_flat, D):
    CWM = 125
    nchunk = _EPS // CWM
    nvr = D // 16

    @functools.partial(
        pl.kernel,
        out_type=jax.ShapeDtypeStruct((_NC * _N, D), _f32),
        mesh=_mesh(),
        compiler_params=_SC_PARAMS,
        scratch_types=[
            pltpu.VMEM((CWM, D), _f32),
            pltpu.VMEM((CWM, D), _f32),
            pltpu.VMEM((nchunk, CWM), _i32),
            pltpu.VMEM((nchunk, CWM), _i32),
            pltpu.VMEM((_EPS,), _f32),
            pltpu.VMEM_SHARED((_N, D), _f32),
            pltpu.SemaphoreType.DMA,
            pltpu.SemaphoreType.DMA,
        ],
    )
    def k(a_hbm, b_hbm, src_hbm, dst_hbm, ex_hbm, out_hbm,
          rows_v, rows_w, src_v, dst_v, ex_v, acc, semA, semB):
        cid = lax.axis_index("c")
        sid = lax.axis_index("s")
        rbase = sid * nchunk
        pltpu.sync_copy(src_hbm.at[pl.ds(rbase, nchunk)], src_v)
        pltpu.sync_copy(dst_hbm.at[pl.ds(rbase, nchunk)], dst_v)
        pltpu.sync_copy(ex_hbm.at[pl.ds(sid * _EPS, _EPS)], ex_v)

        zf = jnp.zeros((16,), _f32)
        def zrow(r, _):
            for kk in range(nvr):
                rows_v[r, pl.ds(kk * 16, 16)] = zf
            return ()
        lax.fori_loop(0, CWM, zrow, ())
        base = sid * 625
        for kk in range(5):
            pltpu.sync_copy(rows_v, acc.at[pl.ds(base + kk * CWM, CWM)])
        plsc.subcore_barrier()

        def sweep(tab_hbm):
            def gather(c, buf, sem):
                pltpu.async_copy(tab_hbm.at[src_v.at[c]], buf, sem)

            def process(c, buf, sem):
                pltpu.make_async_copy(
                    tab_hbm.at[src_v.at[c]], buf, sem).wait()
                def scale(r, _):
                    ev = plsc.load_gather(
                        ex_v, [jnp.full((16,), c * CWM + r, _i32)])
                    for kk in range(nvr):
                        sl = pl.ds(kk * 16, 16)
                        buf[r, sl] = buf[r, sl] * ev
                    return ()
                lax.fori_loop(0, CWM, scale, ())
                pltpu.sync_copy(buf, acc.at[dst_v.at[c]], add=True)

            gather(0, rows_v, semA)
            def pair(i, _):
                c0 = 2 * i
                gather(c0 + 1, rows_w, semB)
                process(c0, rows_v, semA)
                @pl.when(c0 + 2 < nchunk)
                def _():
                    gather(c0 + 2, rows_v, semA)
                process(c0 + 1, rows_w, semB)
                return ()
            lax.fori_loop(0, nchunk // 2, pair, ())

        @pl.when(cid == 0)
        def _():
            sweep(a_hbm)
        @pl.when(cid == 1)
        def _():
            sweep(b_hbm)

        plsc.subcore_barrier()
        pltpu.sync_copy(acc.at[pl.ds(sid * 625, 625)],
                        out_hbm.at[pl.ds(cid * _N + sid * 625, 625)])

    out = k(tab_a, tab_b, src2d.reshape(_EPS * _NS // CWM, CWM),
            dst2d.reshape(_EPS * _NS // CWM, CWM), ex_flat)
    return out.reshape(_NC, _N, D)


def _sc_pq_gather(p_tab, q_tab, src_flat, dst_flat):
    cwc = 40
    nchunk = _EPT // cwc
    bf16 = jnp.bfloat16

    @functools.partial(
        pl.kernel,
        out_type=jax.ShapeDtypeStruct((_E, 512), bf16),
        mesh=_mesh(),
        compiler_params=_SC_PARAMS,
        scratch_types=[
            pltpu.VMEM((cwc, 512), bf16),
            pltpu.VMEM((cwc, 512), bf16),
            pltpu.VMEM((cwc, 512), bf16),
            pltpu.VMEM((cwc, 512), bf16),
            pltpu.VMEM((_EPT,), _i32),
            pltpu.VMEM((_EPT,), _i32),
            pltpu.SemaphoreType.DMA,
            pltpu.SemaphoreType.DMA,
        ],
    )
    def k(p_hbm, q_hbm, src_hbm, dst_hbm, g_hbm,
          pb0, qb0, pb1, qb1, src_v, dst_v, sem0, sem1):
        cid = lax.axis_index("c")
        sid = lax.axis_index("s")
        wid = sid * _NC + cid
        ebase = wid * _EPT
        pltpu.sync_copy(src_hbm.at[pl.ds(ebase, _EPT)], src_v)
        pltpu.sync_copy(dst_hbm.at[pl.ds(ebase, _EPT)], dst_v)

        def gather(c, pbuf, qbuf, sem):
            d1 = pltpu.async_copy(
                p_hbm.at[dst_v.at[pl.ds(c * cwc, cwc)]], pbuf, sem)
            d2 = pltpu.async_copy(
                q_hbm.at[src_v.at[pl.ds(c * cwc, cwc)]], qbuf, sem)
            return d1, d2

        def process(c, pbuf, qbuf, sem):
            pltpu.make_async_copy(
                p_hbm.at[dst_v.at[pl.ds(c * cwc, cwc)]], pbuf, sem).wait()
            pltpu.make_async_copy(
                q_hbm.at[src_v.at[pl.ds(c * cwc, cwc)]], qbuf, sem).wait()
            def addrow(r, _):
                for kk in range(16):
                    sl = pl.ds(kk * 32, 32)
                    pbuf[r, sl] = pbuf[r, sl] + qbuf[r, sl]
                return ()
            lax.fori_loop(0, cwc, addrow, ())
            pltpu.sync_copy(pbuf, g_hbm.at[pl.ds(ebase + c * cwc, cwc)])

        gather(0, pb0, qb0, sem0)
        def pair(i, _):
            c0 = 2 * i
            gather(c0 + 1, pb1, qb1, sem1)
            process(c0, pb0, qb0, sem0)
            @pl.when(c0 + 2 < nchunk)
            def _():
                gather(c0 + 2, pb0, qb0, sem0)
            process(c0 + 1, pb1, qb1, sem1)
            return ()
        lax.fori_loop(0, nchunk // 2, pair, ())

    return k(p_tab, q_tab, src_flat, dst_flat)


def _tc_ab(feat, wac):
    def body(x_ref, w_ref, ab_o, lo_o, hi_o):
        x = x_ref[...]
        ab_o[...] = jnp.dot(x, w_ref[...], preferred_element_type=_f32)
        zpad = jnp.zeros((x.shape[0], 16), _f32)
        lo_o[...] = jnp.concatenate([x[:, :64], zpad], axis=1)
        hi_o[...] = jnp.concatenate([x[:, 64:], zpad], axis=1)
    return pl.pallas_call(
        body,
        grid=(_N // 1000,),
        in_specs=[pl.BlockSpec((1000, 128), lambda i: (i, 0)),
                  pl.BlockSpec((128, 2), lambda i: (0, 0))],
        out_specs=[pl.BlockSpec((1000, 2), lambda i: (i, 0)),
                   pl.BlockSpec((1000, 80), lambda i: (i, 0)),
                   pl.BlockSpec((1000, 80), lambda i: (i, 0))],
        out_shape=[jax.ShapeDtypeStruct((_N, 2), _f32),
                   jax.ShapeDtypeStruct((_N, 80), _f32),
                   jax.ShapeDtypeStruct((_N, 80), _f32)],
    )(feat, wac)


def _tc_node_update(nf_lo, nf_hi, nlq, den, mf, ml01, ml23,
                    fcWa, fcWb, fcb, fclWx, fclWm, fclb, proj_ws, final):
    NB = 400

    def body(nflo_ref, nfhi_ref, q0_ref, q1_ref, q2_ref, q3_ref, den_ref,
             mf_ref, ml01_ref, ml23_ref,
             fcWa_ref, fcWb_ref, fcb_ref, fclWx_ref, fclWm_ref, fclb_ref,
             *rest):
        if final:
            wdf_ref, wdl_ref, wsl_ref, wsf_ref, cb1_ref, p_ref, q_ref = rest
        else:
            (wa_ref, nflo_o, nfhi_o, q0_o, q1_o, q2_o, q3_o, ab_o) = rest
        den = jnp.sum(den_ref[:, pl.program_id(0), :], axis=0) + 1e-9
        inv = (1.0 / den)[:, None]
        mfn = jnp.concatenate([mf_ref[0, :, :64], mf_ref[1, :, :64]],
                              axis=1) * inv
        mln = jnp.concatenate([ml01_ref[0], ml01_ref[1],
                               ml23_ref[0], ml23_ref[1]], axis=1) * inv
        nf_in = jnp.concatenate([nflo_ref[:, :64], nfhi_ref[:, :64]], axis=1)
        nl_in = jnp.concatenate([q0_ref[...], q1_ref[...],
                                 q2_ref[...], q3_ref[...]], axis=1)
        nf2 = jnp.dot(nf_in, fcWa_ref[...], preferred_element_type=_f32)
        nf2 = nf2 + jnp.dot(mfn, fcWb_ref[...], preferred_element_type=_f32)
        nf2 = jnp.maximum(nf2 + fcb_ref[...], 0.0)
        nl2 = jnp.dot(nl_in, fclWx_ref[...], preferred_element_type=_f32)
        nl2 = nl2 + jnp.dot(mln, fclWm_ref[...], preferred_element_type=_f32)
        nl2 = jnp.maximum(nl2 + fclb_ref[...], 0.0)
        if final:
            p = jnp.dot(nf2, wdf_ref[...], preferred_element_type=_f32)
            p = p + jnp.dot(nl2, wdl_ref[...], preferred_element_type=_f32)
            p_ref[...] = (p + cb1_ref[...]).astype(jnp.bfloat16)
            q = jnp.dot(nl2, wsl_ref[...], preferred_element_type=_f32)
            q = q + jnp.dot(nf2, wsf_ref[...], preferred_element_type=_f32)
            q_ref[...] = q.astype(jnp.bfloat16)
        else:
            zpad = jnp.zeros((nf2.shape[0], 16), _f32)
            nflo_o[...] = jnp.concatenate([nf2[:, :64], zpad], axis=1)
            nfhi_o[...] = jnp.concatenate([nf2[:, 64:], zpad], axis=1)
            q0_o[...] = nl2[:, 0:80]
            q1_o[...] = nl2[:, 80:160]
            q2_o[...] = nl2[:, 160:240]
            q3_o[...] = nl2[:, 240:320]
            ab_o[...] = jnp.dot(nf2, wa_ref[...], preferred_element_type=_f32)

    full = lambda shp: pl.BlockSpec(shp, lambda i: tuple(0 for _ in shp))
    in_specs = [
        pl.BlockSpec((NB, 80), lambda i: (i, 0)),
        pl.BlockSpec((NB, 80), lambda i: (i, 0)),
        pl.BlockSpec((NB, 80), lambda i: (i, 0)),
        pl.BlockSpec((NB, 80), lambda i: (i, 0)),
        pl.BlockSpec((NB, 80), lambda i: (i, 0)),
        pl.BlockSpec((NB, 80), lambda i: (i, 0)),
        pl.BlockSpec((_NW, 25, NB), lambda i: (0, 0, 0)),
        pl.BlockSpec((2, NB, 80), lambda i: (0, i, 0)),
        pl.BlockSpec((2, NB, 80), lambda i: (0, i, 0)),
        pl.BlockSpec((2, NB, 80), lambda i: (0, i, 0)),
        full((128, 128)), full((128, 128)), full((1, 128)),
        full((320, 320)), full((320, 320)), full((1, 320)),
    ]
    args = [nf_lo, nf_hi, *nlq, den, mf, ml01, ml23,
            fcWa, fcWb, fcb, fclWx, fclWm, fclb]
    if final:
        wdf, wdl, wsl, wsf, cb1 = proj_ws
        in_specs += [full((128, 512)), full((320, 512)), full((320, 512)),
                     full((128, 512)), full((1, 512))]
        args += [wdf, wdl, wsl, wsf, cb1]
        out_specs = [pl.BlockSpec((NB, 512), lambda i: (i, 0)),
                     pl.BlockSpec((NB, 512), lambda i: (i, 0))]
        out_shape = [jax.ShapeDtypeStruct((_N, 512), jnp.bfloat16),
                     jax.ShapeDtypeStruct((_N, 512), jnp.bfloat16)]
    else:
        wac = proj_ws
        in_specs += [full((128, 2))]
        args += [wac]
        out_specs = [pl.BlockSpec((NB, 80), lambda i: (i, 0)),
                     pl.BlockSpec((NB, 80), lambda i: (i, 0)),
                     pl.BlockSpec((NB, 80), lambda i: (i, 0)),
                     pl.BlockSpec((NB, 80), lambda i: (i, 0)),
                     pl.BlockSpec((NB, 80), lambda i: (i, 0)),
                     pl.BlockSpec((NB, 80), lambda i: (i, 0)),
                     pl.BlockSpec((NB, 2), lambda i: (i, 0))]
        out_shape = [jax.ShapeDtypeStruct((_N, 80), _f32),
                     jax.ShapeDtypeStruct((_N, 80), _f32),
                     jax.ShapeDtypeStruct((_N, 80), _f32),
                     jax.ShapeDtypeStruct((_N, 80), _f32),
                     jax.ShapeDtypeStruct((_N, 80), _f32),
                     jax.ShapeDtypeStruct((_N, 80), _f32),
                     jax.ShapeDtypeStruct((_N, 2), _f32)]

    return pl.pallas_call(
        body,
        grid=(_N // NB,),
        in_specs=in_specs,
        out_specs=out_specs,
        out_shape=out_shape,
    )(*args)


def _tc_readout(g, spatial, wsp, w2, cb2):
    EB = 2000

    def body(g_ref, sp_ref, wsp_ref, w2_ref, cb2_ref, o_ref):
        h = g_ref[...].astype(_f32) + jnp.dot(sp_ref[...], wsp_ref[...],
                                              preferred_element_type=_f32)
        h = jnp.maximum(h, 0.0)
        o_ref[...] = jnp.dot(h, w2_ref[...],
                             preferred_element_type=_f32) + cb2_ref[...]

    return pl.pallas_call(
        body,
        grid=(_E // EB,),
        in_specs=[pl.BlockSpec((EB, 512), lambda i: (i, 0)),
                  pl.BlockSpec((EB, 16), lambda i: (i, 0)),
                  pl.BlockSpec((16, 512), lambda i: (0, 0)),
                  pl.BlockSpec((512, 117), lambda i: (0, 0)),
                  pl.BlockSpec((1, 117), lambda i: (0, 0))],
        out_specs=pl.BlockSpec((EB, 117), lambda i: (i, 0)),
        out_shape=jax.ShapeDtypeStruct((_E, 117), _f32),
    )(g, spatial, wsp, w2, cb2)


def kernel(feat, word2vec, spatial_feat, edge_index, Wa1, Wa2, fc_W, fc_b,
           fcl_W, fcl_b, cls_W1, cls_b1, cls_W2, cls_b2):
    src = edge_index[0]
    dst = edge_index[1]
    src2d = src.reshape(_ERows, _CW)
    dst2d = dst.reshape(_ERows, _CW)

    wa1c = jnp.concatenate([Wa1[:128], Wa1[128:]], axis=1)
    wa2c = jnp.concatenate([Wa2[:128], Wa2[128:]], axis=1)
    fcWa, fcWb = fc_W[:128], fc_W[128:]
    fcb = fc_b.reshape(1, 128)
    fclWx = jnp.pad(fcl_W[:300], ((0, 20), (0, 20)))
    fclWm = jnp.pad(fcl_W[300:], ((0, 20), (0, 20)))
    fclb = jnp.pad(fcl_b, (0, 20)).reshape(1, 320)
    w2v_p = jnp.pad(word2vec, ((0, 0), (0, 20)))
    w2vq = tuple(w2v_p[:, 80 * i:80 * (i + 1)] for i in range(4))
    wdf = cls_W1[0:128]
    wdl = jnp.pad(cls_W1[128:428], ((0, 20), (0, 0)))
    wsp = cls_W1[428:444]
    wsl = jnp.pad(cls_W1[444:744], ((0, 20), (0, 0)))
    wsf = cls_W1[744:872]
    cb1 = cls_b1.reshape(1, 512)
    cb2 = cls_b2.reshape(1, 117)

    ab1, feat_lo, feat_hi = _tc_ab(feat, wa1c)
    ex1, den1 = _sc_scores(ab1, src, dst)
    mf1 = _sc_msg(feat_lo, feat_hi, src2d, dst2d, ex1, 80)
    ml01 = _sc_msg(w2vq[0], w2vq[1], src2d, dst2d, ex1, 80)
    ml23 = _sc_msg(w2vq[2], w2vq[3], src2d, dst2d, ex1, 80)
    nf_lo, nf_hi, q0, q1, q2, q3, ab2 = _tc_node_update(
        feat_lo, feat_hi, w2vq, den1, mf1, ml01, ml23,
        fcWa, fcWb, fcb, fclWx, fclWm, fclb, wa2c, final=False)

    ex2, den2 = _sc_scores(ab2, src, dst)
    mf2 = _sc_msg(nf_lo, nf_hi, src2d, dst2d, ex2, 80)
    ml01_2 = _sc_msg(q0, q1, src2d, dst2d, ex2, 80)
    ml23_2 = _sc_msg(q2, q3, src2d, dst2d, ex2, 80)
    p_tab, q_tab = _tc_node_update(
        nf_lo, nf_hi, (q0, q1, q2, q3), den2, mf2, ml01_2, ml23_2,
        fcWa, fcWb, fcb, fclWx, fclWm, fclb,
        (wdf, wdl, wsl, wsf, cb1), final=True)

    g = _sc_pq_gather(p_tab, q_tab, src, dst)
    return _tc_readout(g, spatial_feat, wsp, cls_W2, cb2)


# speedup vs baseline: 1.1269x; 1.1013x over previous
import functools
import jax
import jax.numpy as jnp
from jax import lax
from jax.experimental import pallas as pl
from jax.experimental.pallas import tpu as pltpu
from jax.experimental.pallas import tpu_sc as plsc

_N = 10000
_E = 320000
_NC = 2
_NS = 16
_NW = _NC * _NS
_CW = 250
_ERows = _E // _CW
_EPT = _E // _NW
_EPS = _E // _NS

_f32 = jnp.float32
_i32 = jnp.int32

_SC_PARAMS = pltpu.CompilerParams(
    needs_layout_passes=False, use_tc_tiling_on_sc=False)


def _mesh():
    return plsc.VectorSubcoreMesh(
        core_axis_name="c", subcore_axis_name="s",
        num_cores=_NC, num_subcores=_NS)


def _sc_scores(ab, src_flat, dst_flat):
    @functools.partial(
        pl.kernel,
        out_type=[jax.ShapeDtypeStruct((_E,), _f32),
                  jax.ShapeDtypeStruct((_NW * _N,), _f32)],
        mesh=_mesh(),
        compiler_params=_SC_PARAMS,
        scratch_types=[
            pltpu.VMEM((2 * _N,), _f32),
            pltpu.VMEM((_EPT,), _i32),
            pltpu.VMEM((_EPT,), _i32),
            pltpu.VMEM((_EPT,), _f32),
            pltpu.VMEM((_N,), _f32),
            pltpu.SemaphoreType.DMA,
        ],
    )
    def k(ab_hbm, src_hbm, dst_hbm, ex_hbm, den_hbm,
          ab_v, src_v, dst_v, ex_v, den_v, sem):
        cid = lax.axis_index("c")
        sid = lax.axis_index("s")
        wid = sid * _NC + cid
        ebase = wid * _EPT
        pltpu.sync_copy(ab_hbm, ab_v)
        pltpu.sync_copy(src_hbm.at[pl.ds(ebase, _EPT)], src_v)
        pltpu.sync_copy(dst_hbm.at[pl.ds(ebase, _EPT)], dst_v)

        zf = jnp.zeros((16,), _f32)
        def zbody(i, _):
            den_v[pl.ds(i * 16, 16)] = zf
            return ()
        lax.fori_loop(0, _N // 16, zbody, ())

        def ebody(g, _):
            s16 = src_v[pl.ds(g * 16, 16)]
            d16 = dst_v[pl.ds(g * 16, 16)]
            a = plsc.load_gather(ab_v, [s16 * 2])
            b = plsc.load_gather(ab_v, [d16 * 2 + 1])
            s = a + b
            s = jnp.where(s >= 0.0, s, s * 0.2)
            e16 = jnp.exp(s)
            ex_v[pl.ds(g * 16, 16)] = e16
            plsc.addupdate_scatter(den_v, [d16], e16)
            return ()
        lax.fori_loop(0, _EPT // 16, ebody, ())

        pltpu.sync_copy(ex_v, ex_hbm.at[pl.ds(ebase, _EPT)])
        pltpu.sync_copy(den_v, den_hbm.at[pl.ds(wid * _N, _N)])

    ex, den = k(ab.reshape(2 * _N), src_flat, dst_flat)
    return ex, den.reshape(_NW, 25, 400)


def _sc_msg(tab_a, tab_b, src2d, dst2d, ex_flat, D):
    CWM = 125
    nchunk = _EPS // CWM
    nvr = D // 16

    @functools.partial(
        pl.kernel,
        out_type=jax.ShapeDtypeStruct((_NC * _N, D), _f32),
        mesh=_mesh(),
        compiler_params=_SC_PARAMS,
        scratch_types=[
            pltpu.VMEM((CWM, D), _f32),
            pltpu.VMEM((CWM, D), _f32),
            pltpu.VMEM((nchunk, CWM), _i32),
            pltpu.VMEM((nchunk, CWM), _i32),
            pltpu.VMEM((_EPS,), _f32),
            pltpu.VMEM_SHARED((_N, D), _f32),
            pltpu.SemaphoreType.DMA,
            pltpu.SemaphoreType.DMA,
        ],
    )
    def k(a_hbm, b_hbm, src_hbm, dst_hbm, ex_hbm, out_hbm,
          rows_v, rows_w, src_v, dst_v, ex_v, acc, semA, semB):
        cid = lax.axis_index("c")
        sid = lax.axis_index("s")
        rbase = sid * nchunk
        pltpu.sync_copy(src_hbm.at[pl.ds(rbase, nchunk)], src_v)
        pltpu.sync_copy(dst_hbm.at[pl.ds(rbase, nchunk)], dst_v)
        pltpu.sync_copy(ex_hbm.at[pl.ds(sid * _EPS, _EPS)], ex_v)

        zf = jnp.zeros((16,), _f32)
        def zrow(r, _):
            for kk in range(nvr):
                rows_v[r, pl.ds(kk * 16, 16)] = zf
            return ()
        lax.fori_loop(0, CWM, zrow, ())
        base = sid * 625
        for kk in range(5):
            pltpu.sync_copy(rows_v, acc.at[pl.ds(base + kk * CWM, CWM)])
        plsc.subcore_barrier()

        def sweep(tab_hbm):
            def gather(c, buf, sem):
                pltpu.async_copy(tab_hbm.at[src_v.at[c]], buf, sem)

            def process(c, buf, sem):
                pltpu.make_async_copy(
                    tab_hbm.at[src_v.at[c]], buf, sem).wait()
                def scale(r, _):
                    ev = plsc.load_gather(
                        ex_v, [jnp.full((16,), c * CWM + r, _i32)])
                    for kk in range(nvr):
                        sl = pl.ds(kk * 16, 16)
                        buf[r, sl] = buf[r, sl] * ev
                    return ()
                lax.fori_loop(0, CWM, scale, ())
                pltpu.sync_copy(buf, acc.at[dst_v.at[c]], add=True)

            gather(0, rows_v, semA)
            def pair(i, _):
                c0 = 2 * i
                gather(c0 + 1, rows_w, semB)
                process(c0, rows_v, semA)
                @pl.when(c0 + 2 < nchunk)
                def _():
                    gather(c0 + 2, rows_v, semA)
                process(c0 + 1, rows_w, semB)
                return ()
            lax.fori_loop(0, nchunk // 2, pair, ())

        @pl.when(cid == 0)
        def _():
            sweep(a_hbm)
        @pl.when(cid == 1)
        def _():
            sweep(b_hbm)

        plsc.subcore_barrier()
        pltpu.sync_copy(acc.at[pl.ds(sid * 625, 625)],
                        out_hbm.at[pl.ds(cid * _N + sid * 625, 625)])

    out = k(tab_a, tab_b, src2d.reshape(_EPS * _NS // CWM, CWM),
            dst2d.reshape(_EPS * _NS // CWM, CWM), ex_flat)
    return out.reshape(_NC, _N, D)


def _sc_pq_gather(p_tab, q_tab, src_flat, dst_flat):
    cwc = 40
    nchunk = _EPT // cwc

    @functools.partial(
        pl.kernel,
        out_type=jax.ShapeDtypeStruct((_E, 512), _f32),
        mesh=_mesh(),
        compiler_params=_SC_PARAMS,
        scratch_types=[
            pltpu.VMEM((cwc, 512), _f32),
            pltpu.VMEM((cwc, 512), _f32),
            pltpu.VMEM((cwc, 512), _f32),
            pltpu.VMEM((cwc, 512), _f32),
            pltpu.VMEM((_EPT,), _i32),
            pltpu.VMEM((_EPT,), _i32),
            pltpu.SemaphoreType.DMA,
            pltpu.SemaphoreType.DMA,
        ],
    )
    def k(p_hbm, q_hbm, src_hbm, dst_hbm, g_hbm,
          pb0, qb0, pb1, qb1, src_v, dst_v, sem0, sem1):
        cid = lax.axis_index("c")
        sid = lax.axis_index("s")
        wid = sid * _NC + cid
        ebase = wid * _EPT
        pltpu.sync_copy(src_hbm.at[pl.ds(ebase, _EPT)], src_v)
        pltpu.sync_copy(dst_hbm.at[pl.ds(ebase, _EPT)], dst_v)

        def gather(c, pbuf, qbuf, sem):
            d1 = pltpu.async_copy(
                p_hbm.at[dst_v.at[pl.ds(c * cwc, cwc)]], pbuf, sem)
            d2 = pltpu.async_copy(
                q_hbm.at[src_v.at[pl.ds(c * cwc, cwc)]], qbuf, sem)
            return d1, d2

        def process(c, pbuf, qbuf, sem):
            pltpu.make_async_copy(
                p_hbm.at[dst_v.at[pl.ds(c * cwc, cwc)]], pbuf, sem).wait()
            pltpu.make_async_copy(
                q_hbm.at[src_v.at[pl.ds(c * cwc, cwc)]], qbuf, sem).wait()
            def addrow(r, _):
                for kk in range(32):
                    sl = pl.ds(kk * 16, 16)
                    pbuf[r, sl] = pbuf[r, sl] + qbuf[r, sl]
                return ()
            lax.fori_loop(0, cwc, addrow, ())
            pltpu.sync_copy(pbuf, g_hbm.at[pl.ds(ebase + c * cwc, cwc)])

        gather(0, pb0, qb0, sem0)
        def pair(i, _):
            c0 = 2 * i
            gather(c0 + 1, pb1, qb1, sem1)
            process(c0, pb0, qb0, sem0)
            @pl.when(c0 + 2 < nchunk)
            def _():
                gather(c0 + 2, pb0, qb0, sem0)
            process(c0 + 1, pb1, qb1, sem1)
            return ()
        lax.fori_loop(0, nchunk // 2, pair, ())

    return k(p_tab, q_tab, src_flat, dst_flat)


def _tc_ab(feat, wac):
    def body(x_ref, w_ref, ab_o, lo_o, hi_o):
        x = x_ref[...]
        ab_o[...] = jnp.dot(x, w_ref[...], preferred_element_type=_f32)
        zpad = jnp.zeros((x.shape[0], 16), _f32)
        lo_o[...] = jnp.concatenate([x[:, :64], zpad], axis=1)
        hi_o[...] = jnp.concatenate([x[:, 64:], zpad], axis=1)
    return pl.pallas_call(
        body,
        grid=(_N // 1000,),
        in_specs=[pl.BlockSpec((1000, 128), lambda i: (i, 0)),
                  pl.BlockSpec((128, 2), lambda i: (0, 0))],
        out_specs=[pl.BlockSpec((1000, 2), lambda i: (i, 0)),
                   pl.BlockSpec((1000, 80), lambda i: (i, 0)),
                   pl.BlockSpec((1000, 80), lambda i: (i, 0))],
        out_shape=[jax.ShapeDtypeStruct((_N, 2), _f32),
                   jax.ShapeDtypeStruct((_N, 80), _f32),
                   jax.ShapeDtypeStruct((_N, 80), _f32)],
    )(feat, wac)


def _tc_node_update(nf_lo, nf_hi, nlq, den, mf, ml01, ml23,
                    fcWa, fcWb, fcb, fclWx, fclWm, fclb, proj_ws, final):
    NB = 400

    def body(nflo_ref, nfhi_ref, q0_ref, q1_ref, q2_ref, q3_ref, den_ref,
             mf_ref, ml01_ref, ml23_ref,
             fcWa_ref, fcWb_ref, fcb_ref, fclWx_ref, fclWm_ref, fclb_ref,
             *rest):
        if final:
            wdf_ref, wdl_ref, wsl_ref, wsf_ref, cb1_ref, p_ref, q_ref = rest
        else:
            (wa_ref, nflo_o, nfhi_o, q0_o, q1_o, q2_o, q3_o, ab_o) = rest
        den = jnp.sum(den_ref[:, pl.program_id(0), :], axis=0) + 1e-9
        inv = (1.0 / den)[:, None]
        mfn = jnp.concatenate([mf_ref[0, :, :64], mf_ref[1, :, :64]],
                              axis=1) * inv
        mln = jnp.concatenate([ml01_ref[0], ml01_ref[1],
                               ml23_ref[0], ml23_ref[1]], axis=1) * inv
        nf_in = jnp.concatenate([nflo_ref[:, :64], nfhi_ref[:, :64]], axis=1)
        nl_in = jnp.concatenate([q0_ref[...], q1_ref[...],
                                 q2_ref[...], q3_ref[...]], axis=1)
        nf2 = jnp.dot(nf_in, fcWa_ref[...], preferred_element_type=_f32)
        nf2 = nf2 + jnp.dot(mfn, fcWb_ref[...], preferred_element_type=_f32)
        nf2 = jnp.maximum(nf2 + fcb_ref[...], 0.0)
        nl2 = jnp.dot(nl_in, fclWx_ref[...], preferred_element_type=_f32)
        nl2 = nl2 + jnp.dot(mln, fclWm_ref[...], preferred_element_type=_f32)
        nl2 = jnp.maximum(nl2 + fclb_ref[...], 0.0)
        if final:
            p = jnp.dot(nf2, wdf_ref[...], preferred_element_type=_f32)
            p = p + jnp.dot(nl2, wdl_ref[...], preferred_element_type=_f32)
            p_ref[...] = p + cb1_ref[...]
            q = jnp.dot(nl2, wsl_ref[...], preferred_element_type=_f32)
            q = q + jnp.dot(nf2, wsf_ref[...], preferred_element_type=_f32)
            q_ref[...] = q
        else:
            zpad = jnp.zeros((nf2.shape[0], 16), _f32)
            nflo_o[...] = jnp.concatenate([nf2[:, :64], zpad], axis=1)
            nfhi_o[...] = jnp.concatenate([nf2[:, 64:], zpad], axis=1)
            q0_o[...] = nl2[:, 0:80]
            q1_o[...] = nl2[:, 80:160]
            q2_o[...] = nl2[:, 160:240]
            q3_o[...] = nl2[:, 240:320]
            ab_o[...] = jnp.dot(nf2, wa_ref[...], preferred_element_type=_f32)

    full = lambda shp: pl.BlockSpec(shp, lambda i: tuple(0 for _ in shp))
    in_specs = [
        pl.BlockSpec((NB, 80), lambda i: (i, 0)),
        pl.BlockSpec((NB, 80), lambda i: (i, 0)),
        pl.BlockSpec((NB, 80), lambda i: (i, 0)),
        pl.BlockSpec((NB, 80), lambda i: (i, 0)),
        pl.BlockSpec((NB, 80), lambda i: (i, 0)),
        pl.BlockSpec((NB, 80), lambda i: (i, 0)),
        pl.BlockSpec((_NW, 25, NB), lambda i: (0, 0, 0)),
        pl.BlockSpec((2, NB, 80), lambda i: (0, i, 0)),
        pl.BlockSpec((2, NB, 80), lambda i: (0, i, 0)),
        pl.BlockSpec((2, NB, 80), lambda i: (0, i, 0)),
        full((128, 128)), full((128, 128)), full((1, 128)),
        full((320, 320)), full((320, 320)), full((1, 320)),
    ]
    args = [nf_lo, nf_hi, *nlq, den, mf, ml01, ml23,
            fcWa, fcWb, fcb, fclWx, fclWm, fclb]
    if final:
        wdf, wdl, wsl, wsf, cb1 = proj_ws
        in_specs += [full((128, 512)), full((320, 512)), full((320, 512)),
                     full((128, 512)), full((1, 512))]
        args += [wdf, wdl, wsl, wsf, cb1]
        out_specs = [pl.BlockSpec((NB, 512), lambda i: (i, 0)),
                     pl.BlockSpec((NB, 512), lambda i: (i, 0))]
        out_shape = [jax.ShapeDtypeStruct((_N, 512), _f32),
                     jax.ShapeDtypeStruct((_N, 512), _f32)]
    else:
        wac = proj_ws
        in_specs += [full((128, 2))]
        args += [wac]
        out_specs = [pl.BlockSpec((NB, 80), lambda i: (i, 0)),
                     pl.BlockSpec((NB, 80), lambda i: (i, 0)),
                     pl.BlockSpec((NB, 80), lambda i: (i, 0)),
                     pl.BlockSpec((NB, 80), lambda i: (i, 0)),
                     pl.BlockSpec((NB, 80), lambda i: (i, 0)),
                     pl.BlockSpec((NB, 80), lambda i: (i, 0)),
                     pl.BlockSpec((NB, 2), lambda i: (i, 0))]
        out_shape = [jax.ShapeDtypeStruct((_N, 80), _f32),
                     jax.ShapeDtypeStruct((_N, 80), _f32),
                     jax.ShapeDtypeStruct((_N, 80), _f32),
                     jax.ShapeDtypeStruct((_N, 80), _f32),
                     jax.ShapeDtypeStruct((_N, 80), _f32),
                     jax.ShapeDtypeStruct((_N, 80), _f32),
                     jax.ShapeDtypeStruct((_N, 2), _f32)]

    return pl.pallas_call(
        body,
        grid=(_N // NB,),
        in_specs=in_specs,
        out_specs=out_specs,
        out_shape=out_shape,
    )(*args)


def _tc_readout(g, spatial, wsp, w2, cb2):
    EB = 2000

    def body(g_ref, sp_ref, wsp_ref, w2_ref, cb2_ref, o_ref):
        h = g_ref[...] + jnp.dot(sp_ref[...], wsp_ref[...],
                                 preferred_element_type=_f32)
        h = jnp.maximum(h, 0.0)
        o_ref[...] = jnp.dot(h, w2_ref[...],
                             preferred_element_type=_f32) + cb2_ref[...]

    return pl.pallas_call(
        body,
        grid=(_E // EB,),
        in_specs=[pl.BlockSpec((EB, 512), lambda i: (i, 0)),
                  pl.BlockSpec((EB, 16), lambda i: (i, 0)),
                  pl.BlockSpec((16, 512), lambda i: (0, 0)),
                  pl.BlockSpec((512, 117), lambda i: (0, 0)),
                  pl.BlockSpec((1, 117), lambda i: (0, 0))],
        out_specs=pl.BlockSpec((EB, 117), lambda i: (i, 0)),
        out_shape=jax.ShapeDtypeStruct((_E, 117), _f32),
    )(g, spatial, wsp, w2, cb2)


def kernel(feat, word2vec, spatial_feat, edge_index, Wa1, Wa2, fc_W, fc_b,
           fcl_W, fcl_b, cls_W1, cls_b1, cls_W2, cls_b2):
    src = edge_index[0]
    dst = edge_index[1]
    src2d = src.reshape(_ERows, _CW)
    dst2d = dst.reshape(_ERows, _CW)

    wa1c = jnp.concatenate([Wa1[:128], Wa1[128:]], axis=1)
    wa2c = jnp.concatenate([Wa2[:128], Wa2[128:]], axis=1)
    fcWa, fcWb = fc_W[:128], fc_W[128:]
    fcb = fc_b.reshape(1, 128)
    fclWx = jnp.pad(fcl_W[:300], ((0, 20), (0, 20)))
    fclWm = jnp.pad(fcl_W[300:], ((0, 20), (0, 20)))
    fclb = jnp.pad(fcl_b, (0, 20)).reshape(1, 320)
    w2v_p = jnp.pad(word2vec, ((0, 0), (0, 20)))
    w2vq = tuple(w2v_p[:, 80 * i:80 * (i + 1)] for i in range(4))
    wdf = cls_W1[0:128]
    wdl = jnp.pad(cls_W1[128:428], ((0, 20), (0, 0)))
    wsp = cls_W1[428:444]
    wsl = jnp.pad(cls_W1[444:744], ((0, 20), (0, 0)))
    wsf = cls_W1[744:872]
    cb1 = cls_b1.reshape(1, 512)
    cb2 = cls_b2.reshape(1, 117)

    ab1, feat_lo, feat_hi = _tc_ab(feat, wa1c)
    ex1, den1 = _sc_scores(ab1, src, dst)
    mf1 = _sc_msg(feat_lo, feat_hi, src2d, dst2d, ex1, 80)
    ml01 = _sc_msg(w2vq[0], w2vq[1], src2d, dst2d, ex1, 80)
    ml23 = _sc_msg(w2vq[2], w2vq[3], src2d, dst2d, ex1, 80)
    nf_lo, nf_hi, q0, q1, q2, q3, ab2 = _tc_node_update(
        feat_lo, feat_hi, w2vq, den1, mf1, ml01, ml23,
        fcWa, fcWb, fcb, fclWx, fclWm, fclb, wa2c, final=False)

    ex2, den2 = _sc_scores(ab2, src, dst)
    mf2 = _sc_msg(nf_lo, nf_hi, src2d, dst2d, ex2, 80)
    ml01_2 = _sc_msg(q0, q1, src2d, dst2d, ex2, 80)
    ml23_2 = _sc_msg(q2, q3, src2d, dst2d, ex2, 80)
    p_tab, q_tab = _tc_node_update(
        nf_lo, nf_hi, (q0, q1, q2, q3), den2, mf2, ml01_2, ml23_2,
        fcWa, fcWb, fcb, fclWx, fclWm, fclb,
        (wdf, wdl, wsl, wsf, cb1), final=True)

    g = _sc_pq_gather(p_tab, q_tab, src, dst)
    return _tc_readout(g, spatial_feat, wsp, cls_W2, cb2)


# speedup vs baseline: 1.1343x; 1.0066x over previous
import functools
import jax
import jax.numpy as jnp
from jax import lax
from jax.experimental import pallas as pl
from jax.experimental.pallas import tpu as pltpu
from jax.experimental.pallas import tpu_sc as plsc

_N = 10000
_E = 320000
_NC = 2
_NS = 16
_NW = _NC * _NS
_CW = 250
_ERows = _E // _CW
_EPT = _E // _NW
_EPS = _E // _NS

_f32 = jnp.float32
_i32 = jnp.int32

_SC_PARAMS = pltpu.CompilerParams(
    needs_layout_passes=False, use_tc_tiling_on_sc=False)


def _mesh():
    return plsc.VectorSubcoreMesh(
        core_axis_name="c", subcore_axis_name="s",
        num_cores=_NC, num_subcores=_NS)


def _sc_scores(ab, src_flat, dst_flat):
    @functools.partial(
        pl.kernel,
        out_type=[jax.ShapeDtypeStruct((_E,), _f32),
                  jax.ShapeDtypeStruct((_NW * _N,), _f32)],
        mesh=_mesh(),
        compiler_params=_SC_PARAMS,
        scratch_types=[
            pltpu.VMEM((2 * _N,), _f32),
            pltpu.VMEM((_EPT,), _i32),
            pltpu.VMEM((_EPT,), _i32),
            pltpu.VMEM((_EPT,), _f32),
            pltpu.VMEM((_N,), _f32),
            pltpu.SemaphoreType.DMA,
        ],
    )
    def k(ab_hbm, src_hbm, dst_hbm, ex_hbm, den_hbm,
          ab_v, src_v, dst_v, ex_v, den_v, sem):
        cid = lax.axis_index("c")
        sid = lax.axis_index("s")
        wid = sid * _NC + cid
        ebase = wid * _EPT
        pltpu.sync_copy(ab_hbm, ab_v)
        pltpu.sync_copy(src_hbm.at[pl.ds(ebase, _EPT)], src_v)
        pltpu.sync_copy(dst_hbm.at[pl.ds(ebase, _EPT)], dst_v)

        zf = jnp.zeros((16,), _f32)
        def zbody(i, _):
            den_v[pl.ds(i * 16, 16)] = zf
            return ()
        lax.fori_loop(0, _N // 16, zbody, ())

        def ebody(g, _):
            s16 = src_v[pl.ds(g * 16, 16)]
            d16 = dst_v[pl.ds(g * 16, 16)]
            a = plsc.load_gather(ab_v, [s16 * 2])
            b = plsc.load_gather(ab_v, [d16 * 2 + 1])
            s = a + b
            s = jnp.where(s >= 0.0, s, s * 0.2)
            e16 = jnp.exp(s)
            ex_v[pl.ds(g * 16, 16)] = e16
            plsc.addupdate_scatter(den_v, [d16], e16)
            return ()
        lax.fori_loop(0, _EPT // 16, ebody, ())

        pltpu.sync_copy(ex_v, ex_hbm.at[pl.ds(ebase, _EPT)])
        pltpu.sync_copy(den_v, den_hbm.at[pl.ds(wid * _N, _N)])

    ex, den = k(ab.reshape(2 * _N), src_flat, dst_flat)
    return ex, den.reshape(_NW, 25, 400)


def _sc_msg(tab_a, tab_b, src2d, dst2d, ex_flat, D):
    CWM = 125
    nchunk = _EPS // CWM
    nvr = D // 16

    @functools.partial(
        pl.kernel,
        out_type=jax.ShapeDtypeStruct((_NC * _N, D), _f32),
        mesh=_mesh(),
        compiler_params=_SC_PARAMS,
        scratch_types=[
            pltpu.VMEM((CWM, D), _f32),
            pltpu.VMEM((CWM, D), _f32),
            pltpu.VMEM((nchunk, CWM), _i32),
            pltpu.VMEM((nchunk, CWM), _i32),
            pltpu.VMEM((_EPS,), _f32),
            pltpu.VMEM_SHARED((_N, D), _f32),
            pltpu.SemaphoreType.DMA,
            pltpu.SemaphoreType.DMA,
        ],
    )
    def k(a_hbm, b_hbm, src_hbm, dst_hbm, ex_hbm, out_hbm,
          rows_v, rows_w, src_v, dst_v, ex_v, acc, semA, semB):
        cid = lax.axis_index("c")
        sid = lax.axis_index("s")
        rbase = sid * nchunk
        pltpu.sync_copy(src_hbm.at[pl.ds(rbase, nchunk)], src_v)
        pltpu.sync_copy(dst_hbm.at[pl.ds(rbase, nchunk)], dst_v)
        pltpu.sync_copy(ex_hbm.at[pl.ds(sid * _EPS, _EPS)], ex_v)

        zf = jnp.zeros((16,), _f32)
        def zrow(r, _):
            for kk in range(nvr):
                rows_v[r, pl.ds(kk * 16, 16)] = zf
            return ()
        lax.fori_loop(0, CWM, zrow, ())
        base = sid * 625
        for kk in range(5):
            pltpu.sync_copy(rows_v, acc.at[pl.ds(base + kk * CWM, CWM)])
        plsc.subcore_barrier()

        def sweep(tab_hbm):
            def gather(c, buf, sem):
                pltpu.async_copy(tab_hbm.at[src_v.at[c]], buf, sem)

            def process(c, buf, sem):
                pltpu.make_async_copy(
                    tab_hbm.at[src_v.at[c]], buf, sem).wait()
                def scale(r, _):
                    ev = plsc.load_gather(
                        ex_v, [jnp.full((16,), c * CWM + r, _i32)])
                    for kk in range(nvr):
                        sl = pl.ds(kk * 16, 16)
                        buf[r, sl] = buf[r, sl] * ev
                    return ()
                lax.fori_loop(0, CWM, scale, ())
                pltpu.sync_copy(buf, acc.at[dst_v.at[c]], add=True)

            gather(0, rows_v, semA)
            def pair(i, _):
                c0 = 2 * i
                gather(c0 + 1, rows_w, semB)
                process(c0, rows_v, semA)
                @pl.when(c0 + 2 < nchunk)
                def _():
                    gather(c0 + 2, rows_v, semA)
                process(c0 + 1, rows_w, semB)
                return ()
            lax.fori_loop(0, nchunk // 2, pair, ())

        @pl.when(cid == 0)
        def _():
            sweep(a_hbm)
        @pl.when(cid == 1)
        def _():
            sweep(b_hbm)

        plsc.subcore_barrier()
        pltpu.sync_copy(acc.at[pl.ds(sid * 625, 625)],
                        out_hbm.at[pl.ds(cid * _N + sid * 625, 625)])

    out = k(tab_a, tab_b, src2d.reshape(_EPS * _NS // CWM, CWM),
            dst2d.reshape(_EPS * _NS // CWM, CWM), ex_flat)
    return out.reshape(_NC, _N, D)


def _sc_pq_gather(p_tab, q_tab, src_half, dst_half):
    EH = _E // 2
    epth = EH // _NW
    cwc = 40
    nchunk = epth // cwc

    @functools.partial(
        pl.kernel,
        out_type=jax.ShapeDtypeStruct((EH, 512), _f32),
        mesh=_mesh(),
        compiler_params=_SC_PARAMS,
        scratch_types=[
            pltpu.VMEM((cwc, 512), _f32),
            pltpu.VMEM((cwc, 512), _f32),
            pltpu.VMEM((cwc, 512), _f32),
            pltpu.VMEM((cwc, 512), _f32),
            pltpu.VMEM((epth,), _i32),
            pltpu.VMEM((epth,), _i32),
            pltpu.SemaphoreType.DMA,
            pltpu.SemaphoreType.DMA,
        ],
    )
    def k(p_hbm, q_hbm, src_hbm, dst_hbm, g_hbm,
          pb0, qb0, pb1, qb1, src_v, dst_v, sem0, sem1):
        cid = lax.axis_index("c")
        sid = lax.axis_index("s")
        wid = sid * _NC + cid
        ebase = wid * epth
        pltpu.sync_copy(src_hbm.at[pl.ds(ebase, epth)], src_v)
        pltpu.sync_copy(dst_hbm.at[pl.ds(ebase, epth)], dst_v)

        def gather(c, pbuf, qbuf, sem):
            pltpu.async_copy(
                p_hbm.at[dst_v.at[pl.ds(c * cwc, cwc)]], pbuf, sem)
            pltpu.async_copy(
                q_hbm.at[src_v.at[pl.ds(c * cwc, cwc)]], qbuf, sem)

        def process(c, pbuf, qbuf, sem):
            pltpu.make_async_copy(
                p_hbm.at[dst_v.at[pl.ds(c * cwc, cwc)]], pbuf, sem).wait()
            pltpu.make_async_copy(
                q_hbm.at[src_v.at[pl.ds(c * cwc, cwc)]], qbuf, sem).wait()
            def addrow(r, _):
                for kk in range(32):
                    sl = pl.ds(kk * 16, 16)
                    pbuf[r, sl] = pbuf[r, sl] + qbuf[r, sl]
                return ()
            lax.fori_loop(0, cwc, addrow, ())
            pltpu.sync_copy(pbuf, g_hbm.at[pl.ds(ebase + c * cwc, cwc)])

        gather(0, pb0, qb0, sem0)
        def pair(i, _):
            c0 = 2 * i
            gather(c0 + 1, pb1, qb1, sem1)
            process(c0, pb0, qb0, sem0)
            @pl.when(c0 + 2 < nchunk)
            def _():
                gather(c0 + 2, pb0, qb0, sem0)
            process(c0 + 1, pb1, qb1, sem1)
            return ()
        lax.fori_loop(0, nchunk // 2, pair, ())
        process(nchunk - 1, pb0, qb0, sem0)

    return k(p_tab, q_tab, src_half, dst_half)


def _tc_ab(feat, wac):
    def body(x_ref, w_ref, ab_o, lo_o, hi_o):
        x = x_ref[...]
        ab_o[...] = jnp.dot(x, w_ref[...], preferred_element_type=_f32)
        zpad = jnp.zeros((x.shape[0], 16), _f32)
        lo_o[...] = jnp.concatenate([x[:, :64], zpad], axis=1)
        hi_o[...] = jnp.concatenate([x[:, 64:], zpad], axis=1)
    return pl.pallas_call(
        body,
        grid=(_N // 1000,),
        in_specs=[pl.BlockSpec((1000, 128), lambda i: (i, 0)),
                  pl.BlockSpec((128, 2), lambda i: (0, 0))],
        out_specs=[pl.BlockSpec((1000, 2), lambda i: (i, 0)),
                   pl.BlockSpec((1000, 80), lambda i: (i, 0)),
                   pl.BlockSpec((1000, 80), lambda i: (i, 0))],
        out_shape=[jax.ShapeDtypeStruct((_N, 2), _f32),
                   jax.ShapeDtypeStruct((_N, 80), _f32),
                   jax.ShapeDtypeStruct((_N, 80), _f32)],
    )(feat, wac)


def _tc_node_update(nf_lo, nf_hi, nlq, den, mf, ml01, ml23,
                    fcWa, fcWb, fcb, fclWx, fclWm, fclb, proj_ws, final):
    NB = 400

    def body(nflo_ref, nfhi_ref, q0_ref, q1_ref, q2_ref, q3_ref, den_ref,
             mf_ref, ml01_ref, ml23_ref,
             fcWa_ref, fcWb_ref, fcb_ref, fclWx_ref, fclWm_ref, fclb_ref,
             *rest):
        if final:
            wdf_ref, wdl_ref, wsl_ref, wsf_ref, cb1_ref, p_ref, q_ref = rest
        else:
            (wa_ref, nflo_o, nfhi_o, q0_o, q1_o, q2_o, q3_o, ab_o) = rest
        den = jnp.sum(den_ref[:, pl.program_id(0), :], axis=0) + 1e-9
        inv = (1.0 / den)[:, None]
        mfn = jnp.concatenate([mf_ref[0, :, :64], mf_ref[1, :, :64]],
                              axis=1) * inv
        mln = jnp.concatenate([ml01_ref[0], ml01_ref[1],
                               ml23_ref[0], ml23_ref[1]], axis=1) * inv
        nf_in = jnp.concatenate([nflo_ref[:, :64], nfhi_ref[:, :64]], axis=1)
        nl_in = jnp.concatenate([q0_ref[...], q1_ref[...],
                                 q2_ref[...], q3_ref[...]], axis=1)
        nf2 = jnp.dot(nf_in, fcWa_ref[...], preferred_element_type=_f32)
        nf2 = nf2 + jnp.dot(mfn, fcWb_ref[...], preferred_element_type=_f32)
        nf2 = jnp.maximum(nf2 + fcb_ref[...], 0.0)
        nl2 = jnp.dot(nl_in, fclWx_ref[...], preferred_element_type=_f32)
        nl2 = nl2 + jnp.dot(mln, fclWm_ref[...], preferred_element_type=_f32)
        nl2 = jnp.maximum(nl2 + fclb_ref[...], 0.0)
        if final:
            p = jnp.dot(nf2, wdf_ref[...], preferred_element_type=_f32)
            p = p + jnp.dot(nl2, wdl_ref[...], preferred_element_type=_f32)
            p_ref[...] = p + cb1_ref[...]
            q = jnp.dot(nl2, wsl_ref[...], preferred_element_type=_f32)
            q = q + jnp.dot(nf2, wsf_ref[...], preferred_element_type=_f32)
            q_ref[...] = q
        else:
            zpad = jnp.zeros((nf2.shape[0], 16), _f32)
            nflo_o[...] = jnp.concatenate([nf2[:, :64], zpad], axis=1)
            nfhi_o[...] = jnp.concatenate([nf2[:, 64:], zpad], axis=1)
            q0_o[...] = nl2[:, 0:80]
            q1_o[...] = nl2[:, 80:160]
            q2_o[...] = nl2[:, 160:240]
            q3_o[...] = nl2[:, 240:320]
            ab_o[...] = jnp.dot(nf2, wa_ref[...], preferred_element_type=_f32)

    full = lambda shp: pl.BlockSpec(shp, lambda i: tuple(0 for _ in shp))
    in_specs = [
        pl.BlockSpec((NB, 80), lambda i: (i, 0)),
        pl.BlockSpec((NB, 80), lambda i: (i, 0)),
        pl.BlockSpec((NB, 80), lambda i: (i, 0)),
        pl.BlockSpec((NB, 80), lambda i: (i, 0)),
        pl.BlockSpec((NB, 80), lambda i: (i, 0)),
        pl.BlockSpec((NB, 80), lambda i: (i, 0)),
        pl.BlockSpec((_NW, 25, NB), lambda i: (0, 0, 0)),
        pl.BlockSpec((2, NB, 80), lambda i: (0, i, 0)),
        pl.BlockSpec((2, NB, 80), lambda i: (0, i, 0)),
        pl.BlockSpec((2, NB, 80), lambda i: (0, i, 0)),
        full((128, 128)), full((128, 128)), full((1, 128)),
        full((320, 320)), full((320, 320)), full((1, 320)),
    ]
    args = [nf_lo, nf_hi, *nlq, den, mf, ml01, ml23,
            fcWa, fcWb, fcb, fclWx, fclWm, fclb]
    if final:
        wdf, wdl, wsl, wsf, cb1 = proj_ws
        in_specs += [full((128, 512)), full((320, 512)), full((320, 512)),
                     full((128, 512)), full((1, 512))]
        args += [wdf, wdl, wsl, wsf, cb1]
        out_specs = [pl.BlockSpec((NB, 512), lambda i: (i, 0)),
                     pl.BlockSpec((NB, 512), lambda i: (i, 0))]
        out_shape = [jax.ShapeDtypeStruct((_N, 512), _f32),
                     jax.ShapeDtypeStruct((_N, 512), _f32)]
    else:
        wac = proj_ws
        in_specs += [full((128, 2))]
        args += [wac]
        out_specs = [pl.BlockSpec((NB, 80), lambda i: (i, 0)),
                     pl.BlockSpec((NB, 80), lambda i: (i, 0)),
                     pl.BlockSpec((NB, 80), lambda i: (i, 0)),
                     pl.BlockSpec((NB, 80), lambda i: (i, 0)),
                     pl.BlockSpec((NB, 80), lambda i: (i, 0)),
                     pl.BlockSpec((NB, 80), lambda i: (i, 0)),
                     pl.BlockSpec((NB, 2), lambda i: (i, 0))]
        out_shape = [jax.ShapeDtypeStruct((_N, 80), _f32),
                     jax.ShapeDtypeStruct((_N, 80), _f32),
                     jax.ShapeDtypeStruct((_N, 80), _f32),
                     jax.ShapeDtypeStruct((_N, 80), _f32),
                     jax.ShapeDtypeStruct((_N, 80), _f32),
                     jax.ShapeDtypeStruct((_N, 80), _f32),
                     jax.ShapeDtypeStruct((_N, 2), _f32)]

    return pl.pallas_call(
        body,
        grid=(_N // NB,),
        in_specs=in_specs,
        out_specs=out_specs,
        out_shape=out_shape,
    )(*args)


def _tc_readout(g, spatial, wsp, w2, cb2):
    EB = 2000
    EH = _E // 2

    def body(g_ref, sp_ref, wsp_ref, w2_ref, cb2_ref, o_ref):
        h = g_ref[...] + jnp.dot(sp_ref[...], wsp_ref[...],
                                 preferred_element_type=_f32)
        h = jnp.maximum(h, 0.0)
        o_ref[...] = jnp.dot(h, w2_ref[...],
                             preferred_element_type=_f32) + cb2_ref[...]

    return pl.pallas_call(
        body,
        grid=(EH // EB,),
        in_specs=[pl.BlockSpec((EB, 512), lambda i: (i, 0)),
                  pl.BlockSpec((EB, 16), lambda i: (i, 0)),
                  pl.BlockSpec((16, 512), lambda i: (0, 0)),
                  pl.BlockSpec((512, 117), lambda i: (0, 0)),
                  pl.BlockSpec((1, 117), lambda i: (0, 0))],
        out_specs=pl.BlockSpec((EB, 117), lambda i: (i, 0)),
        out_shape=jax.ShapeDtypeStruct((EH, 117), _f32),
    )(g, spatial, wsp, w2, cb2)


def kernel(feat, word2vec, spatial_feat, edge_index, Wa1, Wa2, fc_W, fc_b,
           fcl_W, fcl_b, cls_W1, cls_b1, cls_W2, cls_b2):
    src = edge_index[0]
    dst = edge_index[1]
    src2d = src.reshape(_ERows, _CW)
    dst2d = dst.reshape(_ERows, _CW)

    wa1c = jnp.concatenate([Wa1[:128], Wa1[128:]], axis=1)
    wa2c = jnp.concatenate([Wa2[:128], Wa2[128:]], axis=1)
    fcWa, fcWb = fc_W[:128], fc_W[128:]
    fcb = fc_b.reshape(1, 128)
    fclWx = jnp.pad(fcl_W[:300], ((0, 20), (0, 20)))
    fclWm = jnp.pad(fcl_W[300:], ((0, 20), (0, 20)))
    fclb = jnp.pad(fcl_b, (0, 20)).reshape(1, 320)
    w2v_p = jnp.pad(word2vec, ((0, 0), (0, 20)))
    w2vq = tuple(w2v_p[:, 80 * i:80 * (i + 1)] for i in range(4))
    wdf = cls_W1[0:128]
    wdl = jnp.pad(cls_W1[128:428], ((0, 20), (0, 0)))
    wsp = cls_W1[428:444]
    wsl = jnp.pad(cls_W1[444:744], ((0, 20), (0, 0)))
    wsf = cls_W1[744:872]
    cb1 = cls_b1.reshape(1, 512)
    cb2 = cls_b2.reshape(1, 117)

    ab1, feat_lo, feat_hi = _tc_ab(feat, wa1c)
    ex1, den1 = _sc_scores(ab1, src, dst)
    mf1 = _sc_msg(feat_lo, feat_hi, src2d, dst2d, ex1, 80)
    ml01 = _sc_msg(w2vq[0], w2vq[1], src2d, dst2d, ex1, 80)
    ml23 = _sc_msg(w2vq[2], w2vq[3], src2d, dst2d, ex1, 80)
    nf_lo, nf_hi, q0, q1, q2, q3, ab2 = _tc_node_update(
        feat_lo, feat_hi, w2vq, den1, mf1, ml01, ml23,
        fcWa, fcWb, fcb, fclWx, fclWm, fclb, wa2c, final=False)

    ex2, den2 = _sc_scores(ab2, src, dst)
    mf2 = _sc_msg(nf_lo, nf_hi, src2d, dst2d, ex2, 80)
    ml01_2 = _sc_msg(q0, q1, src2d, dst2d, ex2, 80)
    ml23_2 = _sc_msg(q2, q3, src2d, dst2d, ex2, 80)
    p_tab, q_tab = _tc_node_update(
        nf_lo, nf_hi, (q0, q1, q2, q3), den2, mf2, ml01_2, ml23_2,
        fcWa, fcWb, fcb, fclWx, fclWm, fclb,
        (wdf, wdl, wsl, wsf, cb1), final=True)

    preds = []
    for h in range(2):
        sl = slice(h * (_E // 2), (h + 1) * (_E // 2))
        g_h = _sc_pq_gather(p_tab, q_tab, src[sl], dst[sl])
        preds.append(_tc_readout(g_h, spatial_feat[sl], wsp, cls_W2, cb2))
    return jnp.concatenate(preds, axis=0)
